# baseline (speedup 1.0000x reference)
import functools
import math

import jax
import jax.numpy as jnp
from jax.experimental import pallas as pl
from jax.experimental.pallas import tpu as pltpu

N = 4096
D = 1024
H = 1024
O = 1024
NB = 8
R = 8
SCALING = 16.0 / 8.0
TILE = 512

_INV_SQRT2 = 1.0 / math.sqrt(2.0)


def _mlp_kernel(xt_ref, fc1_ref, fc1b_ref, fc2_ref, fc2b_ref,
                a1_ref, b1_ref, a2_ref, b2_ref, out_ref):
    xtf = xt_ref[:, :]
    xt = xtf[:D, :]
    band_row = xtf[D:D + 1, :].astype(jnp.int32)
    T = xtf.shape[1]
    sub_band = jax.lax.broadcasted_iota(jnp.int32, (NB * R, T), 0) // R
    mask_t = (sub_band == band_row).astype(jnp.float32)

    dn_tt = (((0,), (1,)), ((), ()))
    dn_a1 = (((1,), (0,)), ((), ()))
    dn_b = (((0,), (0,)), ((), ()))
    dn_nt = (((1,), (1,)), ((), ()))
    f32 = jnp.float32

    u1t = jax.lax.dot_general(a1_ref[:], xt, dn_a1,
                              preferred_element_type=f32) * mask_t
    h = jax.lax.dot_general(xt, fc1_ref[:], dn_tt,
                            preferred_element_type=f32)
    h += fc1b_ref[:]
    h += jax.lax.dot_general(u1t, b1_ref[:], dn_b,
                             preferred_element_type=f32) * SCALING
    h = 0.5 * h * (1.0 + jax.lax.erf(h * _INV_SQRT2))

    u2t = jax.lax.dot_general(a2_ref[:], h, dn_nt,
                              preferred_element_type=f32) * mask_t
    out = jax.lax.dot_general(h, fc2_ref[:], dn_nt,
                              preferred_element_type=f32)
    out += fc2b_ref[:]
    out += jax.lax.dot_general(u2t, b2_ref[:], dn_b,
                               preferred_element_type=f32) * SCALING
    out_ref[:] = out


@jax.jit
def kernel(x_with_band_info, fc1_w, fc1_b, fc2_w, fc2_b,
           lora_fc1_A, lora_fc1_B, lora_fc2_A, lora_fc2_B):
    a1n = lora_fc1_A.transpose(0, 2, 1).reshape(NB * R, D)
    b1f = lora_fc1_B.reshape(NB * R, H)
    a2n = lora_fc2_A.transpose(0, 2, 1).reshape(NB * R, H)
    b2f = lora_fc2_B.reshape(NB * R, O)
    fc1b = fc1_b.reshape(1, H)
    fc2b = fc2_b.reshape(1, O)
    xt = x_with_band_info.T

    full = lambda shape: pl.BlockSpec(shape, lambda i: (0, 0))
    grid = (N // TILE,)
    out = pl.pallas_call(
        _mlp_kernel,
        grid=grid,
        in_specs=[
            pl.BlockSpec((D + 1, TILE), lambda i: (0, i)),
            full((H, D)),
            full((1, H)),
            full((O, H)),
            full((1, O)),
            full((NB * R, D)),
            full((NB * R, H)),
            full((NB * R, H)),
            full((NB * R, O)),
        ],
        out_specs=pl.BlockSpec((TILE, O), lambda i: (i, 0)),
        out_shape=jax.ShapeDtypeStruct((N, O), jnp.float32),
        compiler_params=pltpu.CompilerParams(
            dimension_semantics=("arbitrary",),
        ),
    )(xt, fc1_w, fc1b, fc2_w, fc2b, a1n, b1f, a2n, b2f)
    return out


# speedup vs baseline: 1.0113x; 1.0113x over previous
import functools
import math

import jax
import jax.numpy as jnp
from jax.experimental import pallas as pl
from jax.experimental.pallas import tpu as pltpu

N = 4096
D = 1024
H = 1024
O = 1024
NB = 8
R = 8
SCALING = 16.0 / 8.0
TILE = 1024

_INV_SQRT2 = 1.0 / math.sqrt(2.0)


def _mlp_kernel(xt_ref, fc1_ref, fc1b_ref, fc2_ref, fc2b_ref,
                a1_ref, b1_ref, a2_ref, b2_ref, out_ref):
    xtf = xt_ref[:, :]
    xt = xtf[:D, :]
    band_row = xtf[D:D + 1, :].astype(jnp.int32)
    T = xtf.shape[1]
    sub_band = jax.lax.broadcasted_iota(jnp.int32, (NB * R, T), 0) // R
    mask_t = (sub_band == band_row).astype(jnp.float32)

    dn_tt = (((0,), (1,)), ((), ()))
    dn_a1 = (((1,), (0,)), ((), ()))
    dn_b = (((0,), (0,)), ((), ()))
    dn_nt = (((1,), (1,)), ((), ()))
    f32 = jnp.float32

    u1t = jax.lax.dot_general(a1_ref[:], xt, dn_a1,
                              preferred_element_type=f32) * mask_t
    h = jax.lax.dot_general(xt, fc1_ref[:], dn_tt,
                            preferred_element_type=f32)
    h += fc1b_ref[:]
    h += jax.lax.dot_general(u1t, b1_ref[:], dn_b,
                             preferred_element_type=f32) * SCALING
    h = 0.5 * h * (1.0 + jax.lax.erf(h * _INV_SQRT2))

    u2t = jax.lax.dot_general(a2_ref[:], h, dn_nt,
                              preferred_element_type=f32) * mask_t
    out = jax.lax.dot_general(h, fc2_ref[:], dn_nt,
                              preferred_element_type=f32)
    out += fc2b_ref[:]
    out += jax.lax.dot_general(u2t, b2_ref[:], dn_b,
                               preferred_element_type=f32) * SCALING
    out_ref[:] = out


@jax.jit
def kernel(x_with_band_info, fc1_w, fc1_b, fc2_w, fc2_b,
           lora_fc1_A, lora_fc1_B, lora_fc2_A, lora_fc2_B):
    a1n = lora_fc1_A.transpose(0, 2, 1).reshape(NB * R, D)
    b1f = lora_fc1_B.reshape(NB * R, H)
    a2n = lora_fc2_A.transpose(0, 2, 1).reshape(NB * R, H)
    b2f = lora_fc2_B.reshape(NB * R, O)
    fc1b = fc1_b.reshape(1, H)
    fc2b = fc2_b.reshape(1, O)
    xt = x_with_band_info.T

    full = lambda shape: pl.BlockSpec(shape, lambda i: (0, 0))
    grid = (N // TILE,)
    out = pl.pallas_call(
        _mlp_kernel,
        grid=grid,
        in_specs=[
            pl.BlockSpec((D + 1, TILE), lambda i: (0, i)),
            full((H, D)),
            full((1, H)),
            full((O, H)),
            full((1, O)),
            full((NB * R, D)),
            full((NB * R, H)),
            full((NB * R, H)),
            full((NB * R, O)),
        ],
        out_specs=pl.BlockSpec((TILE, O), lambda i: (i, 0)),
        out_shape=jax.ShapeDtypeStruct((N, O), jnp.float32),
        compiler_params=pltpu.CompilerParams(
            dimension_semantics=("parallel",),
        ),
    )(xt, fc1_w, fc1b, fc2_w, fc2b, a1n, b1f, a2n, b2f)
    return out
